# baseline (speedup 1.0000x reference)
import functools

import jax
import jax.numpy as jnp
from jax import lax
from jax.experimental import pallas as pl
from jax.experimental.pallas import tpu as pltpu
from jax.experimental.pallas import tpu_sc as plsc

B, S, D, F, E, TOP_K = 2, 2048, 768, 3072, 8, 2
N = B * S
TB = 256
NTB = N // TB
M = 256
G = 10240
NBLK = G // M
NW = 32
TPW = N // NW
CHUNK = 32
LANES = 16


def _router_body(x_ref, gw_ref, gb_ref, meta_ref, carry_ref):
    blk = pl.program_id(0)

    @pl.when(blk == 0)
    def _init():
        carry_ref[...] = jnp.zeros_like(carry_ref)

    xb = x_ref[...]
    scores = jnp.dot(xb, gw_ref[...], preferred_element_type=jnp.float32)
    scores = scores + gb_ref[...]
    probs = jax.nn.softmax(scores, axis=-1)

    cols = lax.broadcasted_iota(jnp.int32, (TB, E), 1)
    p1 = jnp.max(probs, axis=-1, keepdims=True)
    i1 = jnp.min(jnp.where(probs == p1, cols, E), axis=-1, keepdims=True)
    masked = jnp.where(cols == i1, -jnp.inf, probs)
    p2 = jnp.max(masked, axis=-1, keepdims=True)
    i2 = jnp.min(jnp.where(masked == p2, cols, E), axis=-1, keepdims=True)
    c1 = 1.0 / (1.0 + jnp.exp(p2 - p1))
    c2 = 1.0 / (1.0 + jnp.exp(p1 - p2))

    oh1 = (cols == i1).astype(jnp.float32)
    oh2 = (cols == i2).astype(jnp.float32)
    cnt = oh1 + oh2

    rr = lax.broadcasted_iota(jnp.int32, (TB, TB), 0)
    cc = lax.broadcasted_iota(jnp.int32, (TB, TB), 1)
    ltri = (rr > cc).astype(jnp.float32)
    excl = jnp.dot(ltri, cnt, preferred_element_type=jnp.float32)

    carry = carry_ref[...]
    base = excl + carry
    rank1 = jnp.sum(base * oh1, axis=-1, keepdims=True)
    rank2 = jnp.sum(base * oh2, axis=-1, keepdims=True)
    newcarry = carry + jnp.sum(cnt, axis=0, keepdims=True)
    carry_ref[...] = newcarry

    rows = lax.broadcasted_iota(jnp.int32, (TB, E), 0)
    totals = jnp.sum(jnp.where(rows % E == cols, newcarry, 0.0),
                     axis=-1, keepdims=True)

    meta = jnp.where(cols == 0, i1.astype(jnp.float32),
           jnp.where(cols == 1, i2.astype(jnp.float32),
           jnp.where(cols == 2, c1,
           jnp.where(cols == 3, c2,
           jnp.where(cols == 4, rank1,
           jnp.where(cols == 5, rank2,
           jnp.where(cols == 6, totals, 0.0)))))))
    meta_ref[...] = meta[None]


def _run_router(x2, gate_w, gate_b):
    return pl.pallas_call(
        _router_body,
        grid=(NTB,),
        in_specs=[
            pl.BlockSpec((TB, D), lambda b: (b, 0)),
            pl.BlockSpec((D, E), lambda b: (0, 0)),
            pl.BlockSpec((1, E), lambda b: (0, 0)),
        ],
        out_specs=pl.BlockSpec((1, TB, E), lambda b: (b, 0, 0)),
        out_shape=jax.ShapeDtypeStruct((NTB, TB, E), jnp.float32),
        scratch_shapes=[pltpu.VMEM((1, E), jnp.float32)],
        compiler_params=pltpu.CompilerParams(
            dimension_semantics=("arbitrary",)),
    )(x2, gate_w, gate_b.reshape(1, E))


def _pos_chunks(offv, e1v, e2v, r1v, r2v, p1v, p2v):
    for k in range(TPW // LANES):
        sl = pl.ds(k * LANES, LANES)
        p1v[sl] = plsc.load_gather(offv, [e1v[sl]]) + r1v[sl]
        p2v[sl] = plsc.load_gather(offv, [e2v[sl]]) + r2v[sl]


def _dispatch_body(x_hbm, e1_hbm, e2_hbm, r1_hbm, r2_hbm, off_hbm, xs_hbm,
                   xv, e1v, e2v, r1v, r2v, p1v, p2v, offv, sem1, sem2):
    wid = lax.axis_index("s") * 2 + lax.axis_index("c")
    tbase = wid * TPW
    pltpu.sync_copy(x_hbm.at[pl.ds(tbase, TPW)], xv)
    pltpu.sync_copy(e1_hbm.at[pl.ds(tbase, TPW)], e1v)
    pltpu.sync_copy(e2_hbm.at[pl.ds(tbase, TPW)], e2v)
    pltpu.sync_copy(r1_hbm.at[pl.ds(tbase, TPW)], r1v)
    pltpu.sync_copy(r2_hbm.at[pl.ds(tbase, TPW)], r2v)
    pltpu.sync_copy(off_hbm, offv)
    _pos_chunks(offv, e1v, e2v, r1v, r2v, p1v, p2v)
    d1 = pltpu.async_copy(xv, xs_hbm.at[p1v], sem1)
    d2 = pltpu.async_copy(xv, xs_hbm.at[p2v], sem2)
    d1.wait()
    d2.wait()


def _run_dispatch(x2, e1, e2, r1, r2, off16):
    mesh = plsc.VectorSubcoreMesh(core_axis_name="c", subcore_axis_name="s")
    f = functools.partial(
        pl.kernel,
        out_type=jax.ShapeDtypeStruct((G, D), jnp.float32),
        mesh=mesh,
        scratch_types=[
            pltpu.VMEM((TPW, D), jnp.float32),
            pltpu.VMEM((TPW,), jnp.int32),
            pltpu.VMEM((TPW,), jnp.int32),
            pltpu.VMEM((TPW,), jnp.int32),
            pltpu.VMEM((TPW,), jnp.int32),
            pltpu.VMEM((TPW,), jnp.int32),
            pltpu.VMEM((TPW,), jnp.int32),
            pltpu.VMEM((LANES,), jnp.int32),
            pltpu.SemaphoreType.DMA,
            pltpu.SemaphoreType.DMA,
        ],
        compiler_params=pltpu.CompilerParams(needs_layout_passes=False),
    )(_dispatch_body)
    return f(x2, e1, e2, r1, r2, off16)


FB = 1536
NF = F // FB


def _ffn_body(gids_ref, xs_ref, w1_ref, b1_ref, w2_ref, b2_ref, y_ref):
    f = pl.program_id(1)
    x = xs_ref[...].astype(jnp.bfloat16)
    w1b = w1_ref[0].astype(jnp.bfloat16)
    h = jnp.dot(x, w1b, preferred_element_type=jnp.float32)
    h = jnp.maximum(h + b1_ref[0], 0.0).astype(jnp.bfloat16)
    w2b = w2_ref[0].astype(jnp.bfloat16)
    contrib = jnp.dot(h, w2b, preferred_element_type=jnp.float32)

    @pl.when(f == 0)
    def _first():
        y_ref[...] = contrib + b2_ref[0]

    @pl.when(f != 0)
    def _rest():
        y_ref[...] += contrib


def _run_ffn(gids, xs, w1, b1, w2, b2):
    grid_spec = pltpu.PrefetchScalarGridSpec(
        num_scalar_prefetch=1,
        grid=(NBLK, NF),
        in_specs=[
            pl.BlockSpec((M, D), lambda m, f, g: (m, 0)),
            pl.BlockSpec((1, D, FB), lambda m, f, g: (g[m], 0, f)),
            pl.BlockSpec((1, 1, FB), lambda m, f, g: (g[m], 0, f)),
            pl.BlockSpec((1, FB, D), lambda m, f, g: (g[m], f, 0)),
            pl.BlockSpec((1, 1, D), lambda m, f, g: (g[m], 0, 0)),
        ],
        out_specs=pl.BlockSpec((M, D), lambda m, f, g: (m, 0)),
    )
    return pl.pallas_call(
        _ffn_body,
        grid_spec=grid_spec,
        out_shape=jax.ShapeDtypeStruct((G, D), jnp.float32),
        compiler_params=pltpu.CompilerParams(
            dimension_semantics=("arbitrary", "arbitrary")),
    )(gids, xs, w1, b1.reshape(E, 1, F), w2, b2.reshape(E, 1, D))


def _combine_body(y_hbm, e1_hbm, e2_hbm, r1_hbm, r2_hbm, off_hbm,
                  c1_hbm, c2_hbm, out_hbm,
                  e1v, e2v, r1v, r2v, p1v, p2v, offv, c1v, c2v,
                  y1, y2, ob, s1, s2):
    wid = lax.axis_index("s") * 2 + lax.axis_index("c")
    tbase = wid * TPW
    pltpu.sync_copy(e1_hbm.at[pl.ds(tbase, TPW)], e1v)
    pltpu.sync_copy(e2_hbm.at[pl.ds(tbase, TPW)], e2v)
    pltpu.sync_copy(r1_hbm.at[pl.ds(tbase, TPW)], r1v)
    pltpu.sync_copy(r2_hbm.at[pl.ds(tbase, TPW)], r2v)
    pltpu.sync_copy(c1_hbm.at[pl.ds(tbase, TPW)], c1v)
    pltpu.sync_copy(c2_hbm.at[pl.ds(tbase, TPW)], c2v)
    pltpu.sync_copy(off_hbm, offv)
    _pos_chunks(offv, e1v, e2v, r1v, r2v, p1v, p2v)

    for ci in range(TPW // CHUNK):
        d1 = pltpu.async_copy(y_hbm.at[p1v.at[pl.ds(ci * CHUNK, CHUNK)]],
                              y1, s1)
        d2 = pltpu.async_copy(y_hbm.at[p2v.at[pl.ds(ci * CHUNK, CHUNK)]],
                              y2, s2)
        d1.wait()
        d2.wait()

        def tok(i, _):
            lane_i = jnp.zeros((LANES,), jnp.int32) + (ci * CHUNK + i)
            c1s = plsc.load_gather(c1v, [lane_i])
            c2s = plsc.load_gather(c2v, [lane_i])
            for j in range(D // LANES):
                sl = pl.ds(j * LANES, LANES)
                ob[i, sl] = c1s * y1[i, sl] + c2s * y2[i, sl]
            return 0

        lax.fori_loop(0, CHUNK, tok, 0)
        pltpu.sync_copy(ob, out_hbm.at[pl.ds(tbase + ci * CHUNK, CHUNK)])


def _run_combine(y, e1, e2, r1, r2, off16, c1, c2):
    mesh = plsc.VectorSubcoreMesh(core_axis_name="c", subcore_axis_name="s")
    f = functools.partial(
        pl.kernel,
        out_type=jax.ShapeDtypeStruct((N, D), jnp.float32),
        mesh=mesh,
        scratch_types=[
            pltpu.VMEM((TPW,), jnp.int32),
            pltpu.VMEM((TPW,), jnp.int32),
            pltpu.VMEM((TPW,), jnp.int32),
            pltpu.VMEM((TPW,), jnp.int32),
            pltpu.VMEM((TPW,), jnp.int32),
            pltpu.VMEM((TPW,), jnp.int32),
            pltpu.VMEM((LANES,), jnp.int32),
            pltpu.VMEM((TPW,), jnp.float32),
            pltpu.VMEM((TPW,), jnp.float32),
            pltpu.VMEM((CHUNK, D), jnp.float32),
            pltpu.VMEM((CHUNK, D), jnp.float32),
            pltpu.VMEM((CHUNK, D), jnp.float32),
            pltpu.SemaphoreType.DMA,
            pltpu.SemaphoreType.DMA,
        ],
        compiler_params=pltpu.CompilerParams(needs_layout_passes=False),
    )(_combine_body)
    return f(y, e1, e2, r1, r2, off16, c1, c2)


def kernel(x, gate_w, gate_b, w1, b1, w2, b2):
    x2 = x.reshape(N, D)
    meta = _run_router(x2, gate_w, gate_b)

    flat = meta.reshape(N, E)
    e1 = flat[:, 0].astype(jnp.int32)
    e2 = flat[:, 1].astype(jnp.int32)
    c1 = flat[:, 2]
    c2 = flat[:, 3]
    r1 = flat[:, 4].astype(jnp.int32)
    r2 = flat[:, 5].astype(jnp.int32)
    counts = meta[NTB - 1, 0:E, 6].astype(jnp.int32)

    padded = ((counts + (M - 1)) // M) * M
    cum = jnp.cumsum(padded)
    offsets = jnp.concatenate([jnp.zeros((1,), jnp.int32),
                               cum[:-1].astype(jnp.int32)])
    off16 = jnp.concatenate([offsets,
                             jnp.zeros((LANES - E,), jnp.int32)])
    starts = jnp.arange(NBLK, dtype=jnp.int32) * M
    gids = jnp.sum((starts[:, None] >= cum[None, :].astype(jnp.int32))
                   .astype(jnp.int32), axis=1)
    gids = jnp.minimum(gids, E - 1)

    xs = _run_dispatch(x2, e1, e2, r1, r2, off16)
    y = _run_ffn(gids, xs, w1, b1, w2, b2)
    out = _run_combine(y, e1, e2, r1, r2, off16, c1, c2)
    return out.reshape(B, S, D)


# speedup vs baseline: 1.5155x; 1.5155x over previous
import functools

import jax
import jax.numpy as jnp
from jax import lax
from jax.experimental import pallas as pl
from jax.experimental.pallas import tpu as pltpu
from jax.experimental.pallas import tpu_sc as plsc

B, S, D, F, E, TOP_K = 2, 2048, 768, 3072, 8, 2
N = B * S
TB = 256
NTB = N // TB
M = 256
G = 10240
NBLK = G // M
NG = 48
NW = 32
TPW = N // NW
CHUNK = 64
LANES = 16

ME1, ME2, MC1, MC2, MR1, MR2, MCNT = 0, 1, 2, 3, 4, 5, 6


def _router_body(x_ref, gw_ref, gb_ref, meta_ref, carry_ref):
    blk = pl.program_id(0)

    @pl.when(blk == 0)
    def _init():
        carry_ref[...] = jnp.zeros_like(carry_ref)

    xb = x_ref[...]
    scores = jnp.dot(xb, gw_ref[...], preferred_element_type=jnp.float32)
    scores = scores + gb_ref[...]
    probs = jax.nn.softmax(scores, axis=-1)

    cols = lax.broadcasted_iota(jnp.int32, (TB, E), 1)
    p1 = jnp.max(probs, axis=-1, keepdims=True)
    i1 = jnp.min(jnp.where(probs == p1, cols, E), axis=-1, keepdims=True)
    masked = jnp.where(cols == i1, -jnp.inf, probs)
    p2 = jnp.max(masked, axis=-1, keepdims=True)
    i2 = jnp.min(jnp.where(masked == p2, cols, E), axis=-1, keepdims=True)
    c1 = 1.0 / (1.0 + jnp.exp(p2 - p1))
    c2 = 1.0 / (1.0 + jnp.exp(p1 - p2))

    oh1 = (cols == i1).astype(jnp.float32)
    oh2 = (cols == i2).astype(jnp.float32)
    cnt = oh1 + oh2

    rr = lax.broadcasted_iota(jnp.int32, (TB, TB), 0)
    cc = lax.broadcasted_iota(jnp.int32, (TB, TB), 1)
    ltri = (rr > cc).astype(jnp.float32)
    excl = jnp.dot(ltri, cnt, preferred_element_type=jnp.float32)

    carry = carry_ref[...]
    base = excl + carry
    rank1 = jnp.sum(base * oh1, axis=-1, keepdims=True)
    rank2 = jnp.sum(base * oh2, axis=-1, keepdims=True)
    newcarry = carry + jnp.sum(cnt, axis=0, keepdims=True)
    carry_ref[...] = newcarry

    rows = lax.broadcasted_iota(jnp.int32, (TB, E), 0)
    totals = jnp.sum(jnp.where(rows % E == cols, newcarry, 0.0),
                     axis=-1, keepdims=True)

    meta = jnp.where(cols == ME1, i1.astype(jnp.float32),
           jnp.where(cols == ME2, i2.astype(jnp.float32),
           jnp.where(cols == MC1, c1,
           jnp.where(cols == MC2, c2,
           jnp.where(cols == MR1, rank1,
           jnp.where(cols == MR2, rank2,
           jnp.where(cols == MCNT, totals, 0.0)))))))
    meta_ref[...] = meta[None]


def _run_router(x2, gate_w, gate_b):
    return pl.pallas_call(
        _router_body,
        grid=(NTB,),
        in_specs=[
            pl.BlockSpec((TB, D), lambda b: (b, 0)),
            pl.BlockSpec((D, E), lambda b: (0, 0)),
            pl.BlockSpec((1, E), lambda b: (0, 0)),
        ],
        out_specs=pl.BlockSpec((1, TB, E), lambda b: (b, 0, 0)),
        out_shape=jax.ShapeDtypeStruct((NTB, TB, E), jnp.float32),
        scratch_shapes=[pltpu.VMEM((1, E), jnp.float32)],
        compiler_params=pltpu.CompilerParams(
            dimension_semantics=("arbitrary",)),
    )(x2, gate_w, gate_b.reshape(1, E))


def _iota16():
    return lax.iota(jnp.int32, LANES)


def _splat(val):
    return jnp.zeros((LANES,), jnp.int32) + val


def _offsets_from_counts(cv, offv):
    counts = plsc.load_gather(cv, [_iota16(), _splat(MCNT)]).astype(jnp.int32)
    padded = ((counts + (M - 1)) >> 8) << 8
    cum = plsc.cumsum(padded)
    offv[...] = cum - padded
    return cum


def _positions(mv, offv, p1v, p2v):
    for k in range(TPW // LANES):
        sl = pl.ds(k * LANES, LANES)
        ridx = _iota16() + (k * LANES)
        e1 = plsc.load_gather(mv, [ridx, _splat(ME1)]).astype(jnp.int32)
        e2 = plsc.load_gather(mv, [ridx, _splat(ME2)]).astype(jnp.int32)
        r1 = plsc.load_gather(mv, [ridx, _splat(MR1)]).astype(jnp.int32)
        r2 = plsc.load_gather(mv, [ridx, _splat(MR2)]).astype(jnp.int32)
        p1v[sl] = plsc.load_gather(offv, [e1]) + r1
        p2v[sl] = plsc.load_gather(offv, [e2]) + r2


def _dispatch_body(x_hbm, meta_hbm, xs_hbm, gids_hbm,
                   xv, mv, cv, offv, p1v, p2v, gv, semx, sem1, sem2):
    wid = lax.axis_index("s") * 2 + lax.axis_index("c")
    tbase = wid * TPW
    dx = pltpu.async_copy(x_hbm.at[pl.ds(tbase, TPW)], xv, semx)
    pltpu.sync_copy(meta_hbm.at[pl.ds(tbase, TPW)], mv)
    pltpu.sync_copy(meta_hbm.at[pl.ds(N - LANES, LANES)], cv)
    cum = _offsets_from_counts(cv, offv)
    _positions(mv, offv, p1v, p2v)

    for l in range(NG // LANES):
        starts = (_iota16() + (l * LANES)) * M
        acc = jnp.zeros((LANES,), jnp.int32)
        for e in range(E):
            ce = jnp.sum(jnp.where(_iota16() == e, cum, 0))
            acc = acc + jnp.where(starts >= ce, 1, 0)
        gv[pl.ds(l * LANES, LANES)] = jnp.minimum(acc, E - 1)

    @pl.when(wid == 0)
    def _write_gids():
        pltpu.sync_copy(gv, gids_hbm)

    dx.wait()
    d1 = pltpu.async_copy(xv, xs_hbm.at[p1v], sem1)
    d2 = pltpu.async_copy(xv, xs_hbm.at[p2v], sem2)
    d1.wait()
    d2.wait()


def _run_dispatch(x2, meta2):
    mesh = plsc.VectorSubcoreMesh(core_axis_name="c", subcore_axis_name="s")
    f = functools.partial(
        pl.kernel,
        out_type=(jax.ShapeDtypeStruct((G, D), jnp.float32),
                  jax.ShapeDtypeStruct((NG,), jnp.int32)),
        mesh=mesh,
        scratch_types=[
            pltpu.VMEM((TPW, D), jnp.float32),
            pltpu.VMEM((TPW, E), jnp.float32),
            pltpu.VMEM((LANES, E), jnp.float32),
            pltpu.VMEM((LANES,), jnp.int32),
            pltpu.VMEM((TPW,), jnp.int32),
            pltpu.VMEM((TPW,), jnp.int32),
            pltpu.VMEM((NG,), jnp.int32),
            pltpu.SemaphoreType.DMA,
            pltpu.SemaphoreType.DMA,
            pltpu.SemaphoreType.DMA,
        ],
        compiler_params=pltpu.CompilerParams(needs_layout_passes=False),
    )(_dispatch_body)
    return f(x2, meta2)


def _ffn_body(gids_ref, xs_ref, w1_ref, b1_ref, w2_ref, b2_ref, y_ref):
    x = xs_ref[...]
    h = jnp.dot(x, w1_ref[0], preferred_element_type=jnp.float32)
    h = jnp.maximum(h + b1_ref[0], 0.0)
    y = jnp.dot(h, w2_ref[0], preferred_element_type=jnp.float32)
    y_ref[...] = y + b2_ref[0]


def _run_ffn(gids, xs, w1, b1, w2, b2):
    grid_spec = pltpu.PrefetchScalarGridSpec(
        num_scalar_prefetch=1,
        grid=(NBLK,),
        in_specs=[
            pl.BlockSpec((M, D), lambda m, g: (m, 0)),
            pl.BlockSpec((1, D, F), lambda m, g: (g[m], 0, 0)),
            pl.BlockSpec((1, 1, F), lambda m, g: (g[m], 0, 0)),
            pl.BlockSpec((1, F, D), lambda m, g: (g[m], 0, 0)),
            pl.BlockSpec((1, 1, D), lambda m, g: (g[m], 0, 0)),
        ],
        out_specs=pl.BlockSpec((M, D), lambda m, g: (m, 0)),
    )
    return pl.pallas_call(
        _ffn_body,
        grid_spec=grid_spec,
        out_shape=jax.ShapeDtypeStruct((G, D), jnp.float32),
        compiler_params=pltpu.CompilerParams(
            dimension_semantics=("arbitrary",)),
    )(gids, xs, w1, b1.reshape(E, 1, F), w2, b2.reshape(E, 1, D))


def _combine_body(y_hbm, meta_hbm, out_hbm,
                  mv, cv, offv, p1v, p2v, y1, y2, s1, s2):
    wid = lax.axis_index("s") * 2 + lax.axis_index("c")
    tbase = wid * TPW
    pltpu.sync_copy(meta_hbm.at[pl.ds(tbase, TPW)], mv)
    pltpu.sync_copy(meta_hbm.at[pl.ds(N - LANES, LANES)], cv)
    _offsets_from_counts(cv, offv)
    _positions(mv, offv, p1v, p2v)

    for ci in range(TPW // CHUNK):
        d1 = pltpu.async_copy(y_hbm.at[p1v.at[pl.ds(ci * CHUNK, CHUNK)]],
                              y1, s1)
        d2 = pltpu.async_copy(y_hbm.at[p2v.at[pl.ds(ci * CHUNK, CHUNK)]],
                              y2, s2)
        d1.wait()
        d2.wait()

        def tok(i, _):
            c1s = plsc.load_gather(mv, [_splat(ci * CHUNK) + i, _splat(MC1)])
            c2s = plsc.load_gather(mv, [_splat(ci * CHUNK) + i, _splat(MC2)])
            for j in range(D // LANES):
                sl = pl.ds(j * LANES, LANES)
                y1[i, sl] = c1s * y1[i, sl] + c2s * y2[i, sl]
            return 0

        lax.fori_loop(0, CHUNK, tok, 0)
        pltpu.sync_copy(y1, out_hbm.at[pl.ds(tbase + ci * CHUNK, CHUNK)])


def _run_combine(y, meta2):
    mesh = plsc.VectorSubcoreMesh(core_axis_name="c", subcore_axis_name="s")
    f = functools.partial(
        pl.kernel,
        out_type=jax.ShapeDtypeStruct((N, D), jnp.float32),
        mesh=mesh,
        scratch_types=[
            pltpu.VMEM((TPW, E), jnp.float32),
            pltpu.VMEM((LANES, E), jnp.float32),
            pltpu.VMEM((LANES,), jnp.int32),
            pltpu.VMEM((TPW,), jnp.int32),
            pltpu.VMEM((TPW,), jnp.int32),
            pltpu.VMEM((CHUNK, D), jnp.float32),
            pltpu.VMEM((CHUNK, D), jnp.float32),
            pltpu.SemaphoreType.DMA,
            pltpu.SemaphoreType.DMA,
        ],
        compiler_params=pltpu.CompilerParams(needs_layout_passes=False),
    )(_combine_body)
    return f(y, meta2)


def kernel(x, gate_w, gate_b, w1, b1, w2, b2):
    x2 = x.reshape(N, D)
    meta2 = _run_router(x2, gate_w, gate_b).reshape(N, E)
    xs, gids = _run_dispatch(x2, meta2)
    y = _run_ffn(gids, xs, w1, b1, w2, b2)
    out = _run_combine(y, meta2)
    return out.reshape(B, S, D)


# speedup vs baseline: 1.6386x; 1.0812x over previous
import functools

import jax
import jax.numpy as jnp
from jax import lax
from jax.experimental import pallas as pl
from jax.experimental.pallas import tpu as pltpu
from jax.experimental.pallas import tpu_sc as plsc

B, S, D, F, E, TOP_K = 2, 2048, 768, 3072, 8, 2
N = B * S
TB = 256
NTB = N // TB
M = 256
G = 10240
NBLK = G // M
NG = 48
NW = 32
TPW = N // NW
CHUNK = 64
LANES = 16

ME1, ME2, MC1, MC2, MR1, MR2, MCNT = 0, 1, 2, 3, 4, 5, 6


def _router_body(x_ref, gw_ref, gb_ref, meta_ref, carry_ref):
    blk = pl.program_id(0)

    @pl.when(blk == 0)
    def _init():
        carry_ref[...] = jnp.zeros_like(carry_ref)

    xb = x_ref[...]
    scores = jnp.dot(xb, gw_ref[...], preferred_element_type=jnp.float32)
    scores = scores + gb_ref[...]
    probs = jax.nn.softmax(scores, axis=-1)

    cols = lax.broadcasted_iota(jnp.int32, (TB, E), 1)
    p1 = jnp.max(probs, axis=-1, keepdims=True)
    i1 = jnp.min(jnp.where(probs == p1, cols, E), axis=-1, keepdims=True)
    masked = jnp.where(cols == i1, -jnp.inf, probs)
    p2 = jnp.max(masked, axis=-1, keepdims=True)
    i2 = jnp.min(jnp.where(masked == p2, cols, E), axis=-1, keepdims=True)
    c1 = 1.0 / (1.0 + jnp.exp(p2 - p1))
    c2 = 1.0 / (1.0 + jnp.exp(p1 - p2))

    oh1 = (cols == i1).astype(jnp.float32)
    oh2 = (cols == i2).astype(jnp.float32)
    cnt = oh1 + oh2

    rr = lax.broadcasted_iota(jnp.int32, (TB, TB), 0)
    cc = lax.broadcasted_iota(jnp.int32, (TB, TB), 1)
    ltri = (rr > cc).astype(jnp.float32)
    excl = jnp.dot(ltri, cnt, preferred_element_type=jnp.float32)

    carry = carry_ref[...]
    base = excl + carry
    rank1 = jnp.sum(base * oh1, axis=-1, keepdims=True)
    rank2 = jnp.sum(base * oh2, axis=-1, keepdims=True)
    newcarry = carry + jnp.sum(cnt, axis=0, keepdims=True)
    carry_ref[...] = newcarry

    rows = lax.broadcasted_iota(jnp.int32, (TB, E), 0)
    totals = jnp.sum(jnp.where(rows % E == cols, newcarry, 0.0),
                     axis=-1, keepdims=True)

    meta = jnp.where(cols == ME1, i1.astype(jnp.float32),
           jnp.where(cols == ME2, i2.astype(jnp.float32),
           jnp.where(cols == MC1, c1,
           jnp.where(cols == MC2, c2,
           jnp.where(cols == MR1, rank1,
           jnp.where(cols == MR2, rank2,
           jnp.where(cols == MCNT, totals, 0.0)))))))
    meta_ref[...] = meta[None]


def _run_router(x2, gate_w, gate_b):
    return pl.pallas_call(
        _router_body,
        grid=(NTB,),
        in_specs=[
            pl.BlockSpec((TB, D), lambda b: (b, 0)),
            pl.BlockSpec((D, E), lambda b: (0, 0)),
            pl.BlockSpec((1, E), lambda b: (0, 0)),
        ],
        out_specs=pl.BlockSpec((1, TB, E), lambda b: (b, 0, 0)),
        out_shape=jax.ShapeDtypeStruct((NTB, TB, E), jnp.float32),
        scratch_shapes=[pltpu.VMEM((1, E), jnp.float32)],
        compiler_params=pltpu.CompilerParams(
            dimension_semantics=("arbitrary",)),
    )(x2, gate_w, gate_b.reshape(1, E))


def _iota16():
    return lax.iota(jnp.int32, LANES)


def _splat(val):
    return jnp.zeros((LANES,), jnp.int32) + val


def _offsets_from_counts(cv, offv):
    counts = plsc.load_gather(cv, [_iota16(), _splat(MCNT)]).astype(jnp.int32)
    padded = jnp.maximum(((counts + (M - 1)) >> 8) << 8, M)
    cum = plsc.cumsum(padded)
    offv[...] = cum - padded
    return cum


def _positions(mv, offv, p1v, p2v):
    for k in range(TPW // LANES):
        sl = pl.ds(k * LANES, LANES)
        ridx = _iota16() + (k * LANES)
        e1 = plsc.load_gather(mv, [ridx, _splat(ME1)]).astype(jnp.int32)
        e2 = plsc.load_gather(mv, [ridx, _splat(ME2)]).astype(jnp.int32)
        r1 = plsc.load_gather(mv, [ridx, _splat(MR1)]).astype(jnp.int32)
        r2 = plsc.load_gather(mv, [ridx, _splat(MR2)]).astype(jnp.int32)
        p1v[sl] = plsc.load_gather(offv, [e1]) + r1
        p2v[sl] = plsc.load_gather(offv, [e2]) + r2


def _dispatch_body(x_hbm, meta_hbm, xs_hbm, gids_hbm,
                   xv, mv, cv, offv, p1v, p2v, gv, semx, sem1, sem2):
    wid = lax.axis_index("s") * 2 + lax.axis_index("c")
    tbase = wid * TPW
    dx = pltpu.async_copy(x_hbm.at[pl.ds(tbase, TPW)], xv, semx)
    pltpu.sync_copy(meta_hbm.at[pl.ds(tbase, TPW)], mv)
    pltpu.sync_copy(meta_hbm.at[pl.ds(N - LANES, LANES)], cv)
    cum = _offsets_from_counts(cv, offv)
    _positions(mv, offv, p1v, p2v)

    for l in range(NG // LANES):
        starts = (_iota16() + (l * LANES)) * M
        acc = jnp.zeros((LANES,), jnp.int32)
        for e in range(E):
            ce = jnp.sum(jnp.where(_iota16() == e, cum, 0))
            acc = acc + jnp.where(starts >= ce, 1, 0)
        gv[pl.ds(l * LANES, LANES)] = jnp.minimum(acc, E - 1)

    @pl.when(wid == 0)
    def _write_gids():
        pltpu.sync_copy(gv, gids_hbm)

    dx.wait()
    d1 = pltpu.async_copy(xv, xs_hbm.at[p1v], sem1)
    d2 = pltpu.async_copy(xv, xs_hbm.at[p2v], sem2)
    d1.wait()
    d2.wait()


def _run_dispatch(x2, meta2):
    mesh = plsc.VectorSubcoreMesh(core_axis_name="c", subcore_axis_name="s")
    f = functools.partial(
        pl.kernel,
        out_type=(jax.ShapeDtypeStruct((G, D), jnp.float32),
                  jax.ShapeDtypeStruct((NG,), jnp.int32)),
        mesh=mesh,
        scratch_types=[
            pltpu.VMEM((TPW, D), jnp.float32),
            pltpu.VMEM((TPW, E), jnp.float32),
            pltpu.VMEM((LANES, E), jnp.float32),
            pltpu.VMEM((LANES,), jnp.int32),
            pltpu.VMEM((TPW,), jnp.int32),
            pltpu.VMEM((TPW,), jnp.int32),
            pltpu.VMEM((NG,), jnp.int32),
            pltpu.SemaphoreType.DMA,
            pltpu.SemaphoreType.DMA,
            pltpu.SemaphoreType.DMA,
        ],
        compiler_params=pltpu.CompilerParams(needs_layout_passes=False),
    )(_dispatch_body)
    return f(x2, meta2)


def _ffn_body(gids_ref, xs_ref, b1_ref, b2_ref, w1_hbm, w2_hbm, y_ref,
              w1b, w2b, s10, s11, s20, s21):
    m = pl.program_id(0)
    g = gids_ref[m]
    prev = gids_ref[jnp.maximum(m - 1, 0)]
    start = jnp.logical_or(m == 0, g != prev)
    bi = g % 2

    @pl.when(m == 0)
    def _prime():
        pltpu.make_async_copy(w1_hbm.at[0], w1b.at[0], s10).start()
        pltpu.make_async_copy(w2_hbm.at[0], w2b.at[0], s20).start()
        pltpu.make_async_copy(w1_hbm.at[1], w1b.at[1], s11).start()
        pltpu.make_async_copy(w2_hbm.at[1], w2b.at[1], s21).start()

    @pl.when(jnp.logical_and(start, bi == 0))
    def _even():
        pltpu.make_async_copy(w1_hbm.at[g], w1b.at[0], s10).wait()
        pltpu.make_async_copy(w2_hbm.at[g], w2b.at[0], s20).wait()

        @pl.when(jnp.logical_and(g >= 1, g < E - 1))
        def _pf():
            pltpu.make_async_copy(w1_hbm.at[g + 1], w1b.at[1], s11).start()
            pltpu.make_async_copy(w2_hbm.at[g + 1], w2b.at[1], s21).start()

    @pl.when(jnp.logical_and(start, bi == 1))
    def _odd():
        pltpu.make_async_copy(w1_hbm.at[g], w1b.at[1], s11).wait()
        pltpu.make_async_copy(w2_hbm.at[g], w2b.at[1], s21).wait()

        @pl.when(g < E - 1)
        def _pf():
            pltpu.make_async_copy(w1_hbm.at[g + 1], w1b.at[0], s10).start()
            pltpu.make_async_copy(w2_hbm.at[g + 1], w2b.at[0], s20).start()

    x = xs_ref[...]
    h = jnp.dot(x, w1b[bi], preferred_element_type=jnp.float32)
    h = jnp.maximum(h + b1_ref[0], 0.0)
    y = jnp.dot(h, w2b[bi], preferred_element_type=jnp.float32)
    y_ref[...] = y + b2_ref[0]


def _run_ffn(gids, xs, w1, b1, w2, b2):
    grid_spec = pltpu.PrefetchScalarGridSpec(
        num_scalar_prefetch=1,
        grid=(NBLK,),
        in_specs=[
            pl.BlockSpec((M, D), lambda m, g: (m, 0)),
            pl.BlockSpec((1, 1, F), lambda m, g: (g[m], 0, 0)),
            pl.BlockSpec((1, 1, D), lambda m, g: (g[m], 0, 0)),
            pl.BlockSpec(memory_space=pltpu.MemorySpace.HBM),
            pl.BlockSpec(memory_space=pltpu.MemorySpace.HBM),
        ],
        out_specs=pl.BlockSpec((M, D), lambda m, g: (m, 0)),
        scratch_shapes=[
            pltpu.VMEM((2, D, F), jnp.float32),
            pltpu.VMEM((2, F, D), jnp.float32),
            pltpu.SemaphoreType.DMA,
            pltpu.SemaphoreType.DMA,
            pltpu.SemaphoreType.DMA,
            pltpu.SemaphoreType.DMA,
        ],
    )
    return pl.pallas_call(
        _ffn_body,
        grid_spec=grid_spec,
        out_shape=jax.ShapeDtypeStruct((G, D), jnp.float32),
        compiler_params=pltpu.CompilerParams(
            dimension_semantics=("arbitrary",)),
    )(gids, xs, b1.reshape(E, 1, F), b2.reshape(E, 1, D), w1, w2)


def _combine_body(y_hbm, meta_hbm, out_hbm,
                  mv, cv, offv, p1v, p2v, y1, y2, s1, s2):
    wid = lax.axis_index("s") * 2 + lax.axis_index("c")
    tbase = wid * TPW
    pltpu.sync_copy(meta_hbm.at[pl.ds(tbase, TPW)], mv)
    pltpu.sync_copy(meta_hbm.at[pl.ds(N - LANES, LANES)], cv)
    _offsets_from_counts(cv, offv)
    _positions(mv, offv, p1v, p2v)

    for ci in range(TPW // CHUNK):
        d1 = pltpu.async_copy(y_hbm.at[p1v.at[pl.ds(ci * CHUNK, CHUNK)]],
                              y1, s1)
        d2 = pltpu.async_copy(y_hbm.at[p2v.at[pl.ds(ci * CHUNK, CHUNK)]],
                              y2, s2)
        d1.wait()
        d2.wait()

        def tok(i, _):
            c1s = plsc.load_gather(mv, [_splat(ci * CHUNK) + i, _splat(MC1)])
            c2s = plsc.load_gather(mv, [_splat(ci * CHUNK) + i, _splat(MC2)])
            for j in range(D // LANES):
                sl = pl.ds(j * LANES, LANES)
                y1[i, sl] = c1s * y1[i, sl] + c2s * y2[i, sl]
            return 0

        lax.fori_loop(0, CHUNK, tok, 0)
        pltpu.sync_copy(y1, out_hbm.at[pl.ds(tbase + ci * CHUNK, CHUNK)])


def _run_combine(y, meta2):
    mesh = plsc.VectorSubcoreMesh(core_axis_name="c", subcore_axis_name="s")
    f = functools.partial(
        pl.kernel,
        out_type=jax.ShapeDtypeStruct((N, D), jnp.float32),
        mesh=mesh,
        scratch_types=[
            pltpu.VMEM((TPW, E), jnp.float32),
            pltpu.VMEM((LANES, E), jnp.float32),
            pltpu.VMEM((LANES,), jnp.int32),
            pltpu.VMEM((TPW,), jnp.int32),
            pltpu.VMEM((TPW,), jnp.int32),
            pltpu.VMEM((CHUNK, D), jnp.float32),
            pltpu.VMEM((CHUNK, D), jnp.float32),
            pltpu.SemaphoreType.DMA,
            pltpu.SemaphoreType.DMA,
        ],
        compiler_params=pltpu.CompilerParams(needs_layout_passes=False),
    )(_combine_body)
    return f(y, meta2)


def kernel(x, gate_w, gate_b, w1, b1, w2, b2):
    x2 = x.reshape(N, D)
    meta2 = _run_router(x2, gate_w, gate_b).reshape(N, E)
    xs, gids = _run_dispatch(x2, meta2)
    y = _run_ffn(gids, xs, w1, b1, w2, b2)
    out = _run_combine(y, meta2)
    return out.reshape(B, S, D)
